# baseline (speedup 1.0000x reference)
import functools

import jax
import jax.numpy as jnp
from jax import lax
from jax.experimental import pallas as pl
from jax.experimental.pallas import tpu as pltpu
from jax.experimental.pallas import tpu_sc as plsc

N = 1 << 22
ROWS = 2048
COLS = 2048
HALF = 1024
NS = 16
L = 16

RPT = ROWS // NS
CH = 16
NCH = RPT // CH
CHE = CH * COLS
CH2 = 8
NCH2 = RPT // CH2
OE = CH2 * HALF


def _sq_accum_half(buf, base, accs):
    def body(i, a):
        a0, a1, a2, a3 = a
        r = i >> 4
        p = base + (i & 15) * 64
        x0 = buf[r, pl.ds(p, L)]
        x1 = buf[r, pl.ds(p + 16, L)]
        x2 = buf[r, pl.ds(p + 32, L)]
        x3 = buf[r, pl.ds(p + 48, L)]
        return (a0 + x0 * x0, a1 + x1 * x1, a2 + x2 * x2, a3 + x3 * x3)

    return lax.fori_loop(0, CH * 16, body, accs, unroll=4)


_mesh = plsc.VectorSubcoreMesh(core_axis_name="c", subcore_axis_name="s",
                               num_cores=1, num_subcores=NS)


@functools.partial(
    pl.kernel,
    out_type=(
        jax.ShapeDtypeStruct((N // 2,), jnp.float32),
        jax.ShapeDtypeStruct((L,), jnp.float32),
    ),
    mesh=_mesh,
    scratch_types=[
        pltpu.VMEM((CH, COLS), jnp.float32),
        pltpu.VMEM((CH, COLS), jnp.float32),
        pltpu.VMEM((CH2, HALF), jnp.float32),
        pltpu.VMEM((CH2, HALF), jnp.float32),
        pltpu.VMEM((OE,), jnp.float32),
        pltpu.VMEM((OE,), jnp.float32),
        pltpu.VMEM((2, L), jnp.float32),
        pltpu.VMEM((NS, 2, L), jnp.float32),
        pltpu.VMEM((L,), jnp.float32),
        pltpu.VMEM((L,), jnp.float32),
        pltpu.VMEM_SHARED((NS, 2, L), jnp.float32),
        pltpu.SemaphoreType.DMA,
        pltpu.SemaphoreType.DMA,
        pltpu.SemaphoreType.DMA,
        pltpu.SemaphoreType.DMA,
    ],
)
def _sc_measure(psi_hbm, u_hbm, out_hbm, stats_hbm,
                bufa, bufb, ibufa, ibufb, obufa, obufb, part_v, pall_v,
                u_v, stats_v, shared, sema, semb, semoa, semob):
    sid = lax.axis_index("s")
    zero = jnp.zeros((L,), jnp.float32)
    bufs = (bufa, bufb)
    sems = (sema, semb)
    row1 = sid * RPT

    def start_in(c):
        b = c % 2
        return pltpu.async_copy(
            psi_hbm.at[pl.ds(row1 + c * CH, CH), :], bufs[b], sems[b])

    accs = (zero,) * 8
    copies = [start_in(0), None]
    for c in range(NCH):
        b = c % 2
        copies[b].wait()
        if c + 1 < NCH:
            copies[(c + 1) % 2] = start_in(c + 1)
        accs = (_sq_accum_half(bufs[b], 0, accs[:4])
                + _sq_accum_half(bufs[b], HALF, accs[4:]))
    a0 = (accs[0] + accs[1]) + (accs[2] + accs[3])
    a1 = (accs[4] + accs[5]) + (accs[6] + accs[7])

    part_v[0] = a0
    part_v[1] = a1
    pltpu.sync_copy(part_v, shared.at[sid])
    plsc.subcore_barrier()
    pltpu.sync_copy(shared, pall_v)

    def red_body(i, accs):
        a0, a1 = accs
        return (a0 + pall_v[i, 0], a1 + pall_v[i, 1])

    a0, a1 = lax.fori_loop(0, NS, red_body, (zero, zero))
    idx = lax.iota(jnp.int32, L)
    for w in (8, 4, 2, 1):
        a0 = a0 + a0.at[idx ^ w].get(mode="promise_in_bounds")
        a1 = a1 + a1.at[idx ^ w].get(mode="promise_in_bounds")
    s0 = a0[0]
    s1 = a1[0]

    pltpu.sync_copy(u_hbm, u_v)
    u_s = u_v[...][0]

    total = s0 + s1
    outcome = u_s * total > s0
    selected = jnp.where(outcome, s1, s0)
    tb = lax.bitcast_convert_type(total, jnp.int32)
    inv_t = lax.bitcast_convert_type(jnp.int32(0x7EF127EA) - tb, jnp.float32)
    for _ in range(4):
        inv_t = inv_t * (2.0 - total * inv_t)
    p_out = selected * inv_t
    bits = lax.bitcast_convert_type(p_out, jnp.int32)
    y = lax.bitcast_convert_type(jnp.int32(0x5F3759DF) - (bits >> 1),
                                 jnp.float32)
    for _ in range(4):
        y = y * (1.5 - 0.5 * p_out * y * y)
    scale = y

    @pl.when(sid == 0)
    def _():
        outf = jnp.where(outcome, 1.0, 0.0)
        iv = lax.iota(jnp.int32, L)
        stats_v[...] = jnp.where(iv == 0, outf,
                                 jnp.where(iv == 1, p_out, 0.0))
        pltpu.sync_copy(stats_v, stats_hbm)

    off = jnp.where(outcome, HALF, 0)
    psi2 = psi_hbm
    ibufs = (ibufa, ibufb)
    obufs = (obufa, obufb)
    semso = (semoa, semob)
    row_lo = sid * RPT
    obase = sid * RPT * HALF

    def start_in2(c):
        b = c % 2
        return pltpu.async_copy(
            psi2.at[pl.ds(row_lo + c * CH2, CH2), pl.ds(off, HALF)],
            ibufs[b], sems[b])

    def scale_chunk(ibuf, obuf):
        def body(i, carry):
            r = i >> 4
            q = (i & 15) * 64
            o = r * HALF + q
            for k in range(4):
                obuf[pl.ds(o + k * L, L)] = ibuf[r, pl.ds(q + k * L, L)] * scale
            return carry
        lax.fori_loop(0, CH2 * 16, body, 0, unroll=4)

    in_copies = [start_in2(0), start_in2(1)]
    out_copies = [None, None]
    for c in range(NCH2):
        b = c % 2
        in_copies[b].wait()
        if out_copies[b] is not None:
            out_copies[b].wait()
        scale_chunk(ibufs[b], obufs[b])
        out_copies[b] = pltpu.async_copy(
            obufs[b], out_hbm.at[pl.ds(obase + c * OE, OE)], semso[b])
        if c + 2 < NCH2:
            in_copies[b] = start_in2(c + 2)
    out_copies[0].wait()
    out_copies[1].wait()


def kernel(psi, u):
    u16 = jnp.full((L,), u, jnp.float32)
    psi_post, stats = _sc_measure(psi.reshape(ROWS, COLS), u16)
    outcome = stats[0] > 0.5
    p_outcome = stats[1]
    return psi_post, outcome, p_outcome


# speedup vs baseline: 1.4318x; 1.4318x over previous
import functools

import jax
import jax.numpy as jnp
from jax import lax
from jax.experimental import pallas as pl
from jax.experimental.pallas import tpu as pltpu
from jax.experimental.pallas import tpu_sc as plsc

N = 1 << 22
ROWS = 2048
COLS = 2048
HALF = 1024
NC, NS = 2, 16
L = 16

RPT = ROWS // NS
CH = 16
NCH = RPT // CH
CHE = CH * COLS
RPT2 = ROWS // (NC * NS)
NCH2 = RPT2 // CH
OE = CH * HALF


def _sq_accum_half(buf, base, accs):
    def body(i, a):
        a0, a1, a2, a3 = a
        p = (i >> 4) * COLS + base + (i & 15) * 64
        x0 = buf[pl.ds(p, L)]
        x1 = buf[pl.ds(p + 16, L)]
        x2 = buf[pl.ds(p + 32, L)]
        x3 = buf[pl.ds(p + 48, L)]
        return (a0 + x0 * x0, a1 + x1 * x1, a2 + x2 * x2, a3 + x3 * x3)

    return lax.fori_loop(0, CH * 16, body, accs, unroll=4)


_mesh = plsc.VectorSubcoreMesh(core_axis_name="c", subcore_axis_name="s",
                               num_cores=NC, num_subcores=NS)


@functools.partial(
    pl.kernel,
    out_type=(
        jax.ShapeDtypeStruct((N // 2,), jnp.float32),
        jax.ShapeDtypeStruct((L,), jnp.float32),
    ),
    mesh=_mesh,
    scratch_types=[
        pltpu.VMEM((CHE,), jnp.float32),
        pltpu.VMEM((CHE,), jnp.float32),
        pltpu.VMEM((OE,), jnp.float32),
        pltpu.VMEM((OE,), jnp.float32),
        pltpu.VMEM((2, L), jnp.float32),
        pltpu.VMEM((NS, 2, L), jnp.float32),
        pltpu.VMEM((L,), jnp.float32),
        pltpu.VMEM((L,), jnp.float32),
        pltpu.VMEM_SHARED((NS, 2, L), jnp.float32),
        pltpu.SemaphoreType.DMA,
        pltpu.SemaphoreType.DMA,
        pltpu.SemaphoreType.DMA,
        pltpu.SemaphoreType.DMA,
    ],
)
def _sc_measure(psi_hbm, u_hbm, out_hbm, stats_hbm,
                bufa, bufb, obufa, obufb, part_v, pall_v, u_v, stats_v,
                shared, sema, semb, semoa, semob):
    cid = lax.axis_index("c")
    sid = lax.axis_index("s")
    zero = jnp.zeros((L,), jnp.float32)
    bufs = (bufa, bufb)
    sems = (sema, semb)

    def start_in(base_el, c):
        b = c % 2
        return pltpu.async_copy(
            psi_hbm.at[pl.ds(base_el + c * CHE, CHE)], bufs[b], sems[b])

    base1 = sid * RPT * COLS
    accs = (zero,) * 8
    copies = [start_in(base1, 0), None]
    for c in range(NCH):
        b = c % 2
        copies[b].wait()
        if c + 1 < NCH:
            copies[(c + 1) % 2] = start_in(base1, c + 1)
        accs = (_sq_accum_half(bufs[b], 0, accs[:4])
                + _sq_accum_half(bufs[b], HALF, accs[4:]))
    a0 = (accs[0] + accs[1]) + (accs[2] + accs[3])
    a1 = (accs[4] + accs[5]) + (accs[6] + accs[7])

    part_v[0] = a0
    part_v[1] = a1
    pltpu.sync_copy(part_v, shared.at[sid])
    plsc.subcore_barrier()
    pltpu.sync_copy(shared, pall_v)

    def red_body(i, accs):
        a0, a1 = accs
        return (a0 + pall_v[i, 0], a1 + pall_v[i, 1])

    a0, a1 = lax.fori_loop(0, NS, red_body, (zero, zero))
    idx = lax.iota(jnp.int32, L)
    for w in (8, 4, 2, 1):
        a0 = a0 + a0.at[idx ^ w].get(mode="promise_in_bounds")
        a1 = a1 + a1.at[idx ^ w].get(mode="promise_in_bounds")
    s0 = a0[0]
    s1 = a1[0]

    pltpu.sync_copy(u_hbm, u_v)
    u_s = u_v[...][0]

    total = s0 + s1
    outcome = u_s * total > s0
    selected = jnp.where(outcome, s1, s0)
    tb = lax.bitcast_convert_type(total, jnp.int32)
    inv_t = lax.bitcast_convert_type(jnp.int32(0x7EF127EA) - tb, jnp.float32)
    for _ in range(4):
        inv_t = inv_t * (2.0 - total * inv_t)
    p_out = selected * inv_t
    bits = lax.bitcast_convert_type(p_out, jnp.int32)
    y = lax.bitcast_convert_type(jnp.int32(0x5F3759DF) - (bits >> 1),
                                 jnp.float32)
    for _ in range(4):
        y = y * (1.5 - 0.5 * p_out * y * y)
    scale = y

    @pl.when(jnp.logical_and(cid == 0, sid == 0))
    def _():
        outf = jnp.where(outcome, 1.0, 0.0)
        iv = lax.iota(jnp.int32, L)
        stats_v[...] = jnp.where(iv == 0, outf,
                                 jnp.where(iv == 1, p_out, 0.0))
        pltpu.sync_copy(stats_v, stats_hbm)

    off = jnp.where(outcome, HALF, 0)
    obufs = (obufa, obufb)
    semso = (semoa, semob)
    wid = cid * NS + sid
    base2 = wid * RPT2 * COLS
    obase = wid * RPT2 * HALF

    def scale_chunk(buf, obuf):
        def body(i, carry):
            r = i >> 4
            q = (i & 15) * 64
            p = r * COLS + off + q
            o = r * HALF + q
            for k in range(4):
                obuf[pl.ds(o + k * L, L)] = buf[pl.ds(p + k * L, L)] * scale
            return carry
        lax.fori_loop(0, CH * 16, body, 0, unroll=4)

    in_copies = [start_in(base2, 0), start_in(base2, 1)]
    out_copies = [None, None]
    for c in range(NCH2):
        b = c % 2
        in_copies[b].wait()
        if out_copies[b] is not None:
            out_copies[b].wait()
        scale_chunk(bufs[b], obufs[b])
        out_copies[b] = pltpu.async_copy(
            obufs[b], out_hbm.at[pl.ds(obase + c * OE, OE)], semso[b])
        if c + 2 < NCH2:
            in_copies[b] = start_in(base2, c + 2)
    out_copies[0].wait()
    out_copies[1].wait()


def kernel(psi, u):
    u16 = jnp.full((L,), u, jnp.float32)
    psi_post, stats = _sc_measure(psi, u16)
    outcome = stats[0] > 0.5
    p_outcome = stats[1]
    return psi_post, outcome, p_outcome


# speedup vs baseline: 1.4401x; 1.0058x over previous
import functools

import jax
import jax.numpy as jnp
from jax import lax
from jax.experimental import pallas as pl
from jax.experimental.pallas import tpu as pltpu
from jax.experimental.pallas import tpu_sc as plsc

N = 1 << 22
ROWS = 2048
COLS = 2048
HALF = 1024
NC, NS = 2, 16
L = 16

RPT = ROWS // NS
CH = 16
NCH = RPT // CH
CHE = CH * COLS
RPT2 = ROWS // (NC * NS)
NCH2 = RPT2 // CH
OE = CH * HALF


def _sq_accum_half(buf, base, accs):
    def body(i, a):
        a0, a1, a2, a3 = a
        p = (i >> 4) * COLS + base + (i & 15) * 64
        x0 = buf[pl.ds(p, L)]
        x1 = buf[pl.ds(p + 16, L)]
        x2 = buf[pl.ds(p + 32, L)]
        x3 = buf[pl.ds(p + 48, L)]
        return (a0 + x0 * x0, a1 + x1 * x1, a2 + x2 * x2, a3 + x3 * x3)

    return lax.fori_loop(0, CH * 16, body, accs, unroll=4)


_mesh = plsc.VectorSubcoreMesh(core_axis_name="c", subcore_axis_name="s",
                               num_cores=NC, num_subcores=NS)


@functools.partial(
    pl.kernel,
    out_type=(
        jax.ShapeDtypeStruct((N // 2,), jnp.float32),
        jax.ShapeDtypeStruct((L,), jnp.float32),
    ),
    mesh=_mesh,
    scratch_types=[
        pltpu.VMEM((CHE,), jnp.float32),
        pltpu.VMEM((CHE,), jnp.float32),
        pltpu.VMEM((OE,), jnp.float32),
        pltpu.VMEM((OE,), jnp.float32),
        pltpu.VMEM((2, L), jnp.float32),
        pltpu.VMEM((NS, 2, L), jnp.float32),
        pltpu.VMEM((L,), jnp.float32),
        pltpu.VMEM((L,), jnp.float32),
        pltpu.VMEM_SHARED((NS, 2, L), jnp.float32),
        pltpu.SemaphoreType.DMA,
        pltpu.SemaphoreType.DMA,
        pltpu.SemaphoreType.DMA,
        pltpu.SemaphoreType.DMA,
    ],
)
def _sc_measure(psi_hbm, u_hbm, out_hbm, stats_hbm,
                bufa, bufb, obufa, obufb, part_v, pall_v, u_v, stats_v,
                shared, sema, semb, semoa, semob):
    cid = lax.axis_index("c")
    sid = lax.axis_index("s")
    zero = jnp.zeros((L,), jnp.float32)
    bufs = (bufa, bufb)
    sems = (sema, semb)

    def start_in(base_el, c):
        b = c % 2
        return pltpu.async_copy(
            psi_hbm.at[pl.ds(base_el + c * CHE, CHE)], bufs[b], sems[b])

    base1 = sid * RPT * COLS
    accs = (zero,) * 8
    copies = [start_in(base1, 0), None]
    for c in range(NCH):
        b = c % 2
        copies[b].wait()
        if c + 1 < NCH:
            copies[(c + 1) % 2] = start_in(base1, c + 1)
        accs = (_sq_accum_half(bufs[b], 0, accs[:4])
                + _sq_accum_half(bufs[b], HALF, accs[4:]))
    a0 = (accs[0] + accs[1]) + (accs[2] + accs[3])
    a1 = (accs[4] + accs[5]) + (accs[6] + accs[7])

    part_v[0] = a0
    part_v[1] = a1
    pltpu.sync_copy(part_v, shared.at[sid])
    plsc.subcore_barrier()
    pltpu.sync_copy(shared, pall_v)

    def red_body(i, accs):
        a0, a1 = accs
        return (a0 + pall_v[i, 0], a1 + pall_v[i, 1])

    a0, a1 = lax.fori_loop(0, NS, red_body, (zero, zero))
    idx = lax.iota(jnp.int32, L)
    for w in (8, 4, 2, 1):
        a0 = a0 + a0.at[idx ^ w].get(mode="promise_in_bounds")
        a1 = a1 + a1.at[idx ^ w].get(mode="promise_in_bounds")
    s0 = a0[0]
    s1 = a1[0]

    pltpu.sync_copy(u_hbm, u_v)
    u_s = u_v[...][0]

    total = s0 + s1
    outcome = u_s * total > s0
    selected = jnp.where(outcome, s1, s0)
    tb = lax.bitcast_convert_type(total, jnp.int32)
    inv_t = lax.bitcast_convert_type(jnp.int32(0x7EF127EA) - tb, jnp.float32)
    for _ in range(4):
        inv_t = inv_t * (2.0 - total * inv_t)
    p_out = selected * inv_t
    bits = lax.bitcast_convert_type(p_out, jnp.int32)
    y = lax.bitcast_convert_type(jnp.int32(0x5F3759DF) - (bits >> 1),
                                 jnp.float32)
    for _ in range(4):
        y = y * (1.5 - 0.5 * p_out * y * y)
    scale = y

    @pl.when(jnp.logical_and(cid == 0, sid == 0))
    def _():
        outf = jnp.where(outcome, 1.0, 0.0)
        iv = lax.iota(jnp.int32, L)
        stats_v[...] = jnp.where(iv == 0, outf,
                                 jnp.where(iv == 1, p_out, 0.0))
        pltpu.sync_copy(stats_v, stats_hbm)

    off = jnp.where(outcome, HALF, 0)
    obufs = (obufa, obufb)
    semso = (semoa, semob)
    wid = cid * NS + sid
    base2 = wid * RPT2 * COLS
    obase = wid * RPT2 * HALF

    row2 = wid * RPT2

    def start_in2(c):
        b = c % 2
        return [
            pltpu.async_copy(
                psi_hbm.at[pl.ds((row2 + c * CH + r) * COLS + off, HALF)],
                bufs[b].at[pl.ds(r * HALF, HALF)], sems[b])
            for r in range(CH)
        ]

    def scale_chunk(buf, obuf):
        def body(i, carry):
            q = i * 64
            for k in range(4):
                obuf[pl.ds(q + k * L, L)] = buf[pl.ds(q + k * L, L)] * scale
            return carry
        lax.fori_loop(0, CH * 16, body, 0, unroll=4)

    in_copies = [start_in2(0), start_in2(1)]
    out_copies = [None, None]
    for c in range(NCH2):
        b = c % 2
        for cp in in_copies[b]:
            cp.wait()
        if out_copies[b] is not None:
            out_copies[b].wait()
        scale_chunk(bufs[b], obufs[b])
        out_copies[b] = pltpu.async_copy(
            obufs[b], out_hbm.at[pl.ds(obase + c * OE, OE)], semso[b])
        if c + 2 < NCH2:
            in_copies[b] = start_in2(c + 2)
    out_copies[0].wait()
    out_copies[1].wait()


def kernel(psi, u):
    u16 = jnp.full((L,), u, jnp.float32)
    psi_post, stats = _sc_measure(psi, u16)
    outcome = stats[0] > 0.5
    p_outcome = stats[1]
    return psi_post, outcome, p_outcome


# speedup vs baseline: 1.5037x; 1.0441x over previous
import functools

import jax
import jax.numpy as jnp
from jax import lax
from jax.experimental import pallas as pl
from jax.experimental.pallas import tpu as pltpu
from jax.experimental.pallas import tpu_sc as plsc

N = 1 << 22
ROWS = 2048
COLS = 2048
HALF = 1024
NC, NS = 2, 16
L = 16

TCG = 32
TCB = N // TCG


def _tc_reduce_body(u_ref, psi_ref, stats_ref, acc_ref):
    i = pl.program_id(0)

    @pl.when(i == 0)
    def _():
        acc_ref[0] = 0.0
        acc_ref[1] = 0.0

    x = psi_ref[...].reshape(TCB // COLS, COLS)
    s0 = jnp.sum(x[:, :HALF] * x[:, :HALF])
    s1 = jnp.sum(x[:, HALF:] * x[:, HALF:])
    acc_ref[0] += s0
    acc_ref[1] += s1

    @pl.when(i == TCG - 1)
    def _():
        t0 = acc_ref[0]
        t1 = acc_ref[1]
        total = t0 + t1
        p0 = t0 / total
        outcome = u_ref[0] > p0
        p_out = jnp.where(outcome, 1.0 - p0, p0)
        scale = lax.rsqrt(p_out)
        outf = jnp.where(outcome, 1.0, 0.0)
        iv = lax.iota(jnp.float32, 128)
        iv_i = lax.iota(jnp.int32, 128)
        del iv
        stats_ref[...] = jnp.where(
            iv_i == 0, outf,
            jnp.where(iv_i == 1, p_out,
                      jnp.where(iv_i == 2, scale, 0.0)))


_tc_reduce = pl.pallas_call(
    _tc_reduce_body,
    grid=(TCG,),
    in_specs=[
        pl.BlockSpec(memory_space=pltpu.SMEM),
        pl.BlockSpec((TCB,), lambda i: (i,)),
    ],
    out_specs=pl.BlockSpec((128,), lambda i: (0,)),
    out_shape=jax.ShapeDtypeStruct((128,), jnp.float32),
    scratch_shapes=[pltpu.SMEM((2,), jnp.float32)],
)

RPT2 = ROWS // (NC * NS)
CH = 16
NCH2 = RPT2 // CH
OE = CH * HALF

_mesh = plsc.VectorSubcoreMesh(core_axis_name="c", subcore_axis_name="s",
                               num_cores=NC, num_subcores=NS)


@functools.partial(
    pl.kernel,
    out_type=jax.ShapeDtypeStruct((N // 2,), jnp.float32),
    mesh=_mesh,
    scratch_types=[
        pltpu.VMEM((OE,), jnp.float32),
        pltpu.VMEM((OE,), jnp.float32),
        pltpu.VMEM((OE,), jnp.float32),
        pltpu.VMEM((OE,), jnp.float32),
        pltpu.VMEM((L,), jnp.float32),
        pltpu.SemaphoreType.DMA,
        pltpu.SemaphoreType.DMA,
        pltpu.SemaphoreType.DMA,
        pltpu.SemaphoreType.DMA,
    ],
)
def _sc_select(psi_hbm, stats_hbm, out_hbm,
               bufa, bufb, obufa, obufb, st_v, sema, semb, semoa, semob):
    cid = lax.axis_index("c")
    sid = lax.axis_index("s")
    bufs = (bufa, bufb)
    obufs = (obufa, obufb)
    sems = (sema, semb)
    semso = (semoa, semob)

    pltpu.sync_copy(stats_hbm.at[pl.ds(0, L)], st_v)
    st = st_v[...]
    outcome = st[0] > 0.5
    scale = st[2]
    off = jnp.where(outcome, HALF, 0)

    wid = cid * NS + sid
    row2 = wid * RPT2
    obase = wid * RPT2 * HALF

    def start_in(c):
        b = c % 2
        return [
            pltpu.async_copy(
                psi_hbm.at[pl.ds((row2 + c * CH + r) * COLS + off, HALF)],
                bufs[b].at[pl.ds(r * HALF, HALF)], sems[b])
            for r in range(CH)
        ]

    def scale_chunk(buf, obuf):
        def body(i, carry):
            q = i * 64
            for k in range(4):
                obuf[pl.ds(q + k * L, L)] = buf[pl.ds(q + k * L, L)] * scale
            return carry
        lax.fori_loop(0, CH * 16, body, 0, unroll=4)

    in_copies = [start_in(0), start_in(1)]
    out_copies = [None, None]
    for c in range(NCH2):
        b = c % 2
        for cp in in_copies[b]:
            cp.wait()
        if out_copies[b] is not None:
            out_copies[b].wait()
        scale_chunk(bufs[b], obufs[b])
        out_copies[b] = pltpu.async_copy(
            obufs[b], out_hbm.at[pl.ds(obase + c * OE, OE)], semso[b])
        if c + 2 < NCH2:
            in_copies[b] = start_in(c + 2)
    out_copies[0].wait()
    out_copies[1].wait()


def kernel(psi, u):
    u1 = jnp.full((1,), u, jnp.float32)
    stats = _tc_reduce(u1, psi)
    psi_post = _sc_select(psi, stats)
    outcome = stats[0] > 0.5
    p_outcome = stats[1]
    return psi_post, outcome, p_outcome
